# baseline (speedup 1.0000x reference)
import functools

import jax
import jax.numpy as jnp
from jax.experimental import pallas as pl
from jax.experimental.pallas import tpu as pltpu

B, T, D = 64, 576, 768
E, R = 8, 192
SPP = 4


def _fused_kernel(tok_ref, gw_ref, gb_ref, wd_ref, bd_ref, wu_ref, bu_ref,
                  out_ref, logits_ref, sel_ref, ew_ref, imp_ref, load_ref):
    g = pl.program_id(0)
    lane = jax.lax.broadcasted_iota(jnp.int32, (1, E), 1)
    imp_acc = jnp.zeros((1, E), jnp.float32)
    load_acc = jnp.zeros((1, E), jnp.float32)
    ones_row = jnp.ones((1, T), jnp.float32)

    for i in range(SPP):
        x = tok_ref[i]

        psum = jax.lax.dot_general(
            ones_row, x, (((1,), (0,)), ((), ())),
            precision=jax.lax.Precision.HIGHEST,
            preferred_element_type=jnp.float32)
        logits = jax.lax.dot_general(
            psum, gw_ref[...], (((1,), (0,)), ((), ())),
            precision=jax.lax.Precision.HIGHEST,
            preferred_element_type=jnp.float32) * (1.0 / T) + gb_ref[...]
        m = jnp.max(logits, axis=-1, keepdims=True)
        ex = jnp.exp(logits - m)
        probs = ex / jnp.sum(ex, axis=-1, keepdims=True)
        top1 = jnp.max(probs)
        sel = jnp.argmax(probs, axis=-1)[0].astype(jnp.int32)

        wd = wd_ref[sel]
        wu = wu_ref[sel]
        bd = bd_ref[sel]
        bu = bu_ref[sel]
        xb = x.astype(jnp.bfloat16)
        h = jnp.dot(xb, wd, preferred_element_type=jnp.float32) + bd
        h = jax.nn.gelu(h.astype(jnp.bfloat16))
        y = jnp.dot(h, wu, preferred_element_type=jnp.float32) + bu
        out_ref[i] = top1 * (x + y)

        logits_ref[i] = logits
        sel_ref[i] = jnp.zeros((1, E), jnp.int32) + sel
        onehot = (lane == sel).astype(jnp.float32)
        ew_ref[i] = onehot * top1
        imp_acc += onehot * top1
        load_acc += onehot * (1.0 / B)

    @pl.when(g == 0)
    def _init():
        imp_ref[...] = jnp.zeros_like(imp_ref)
        load_ref[...] = jnp.zeros_like(load_ref)

    imp_ref[...] += imp_acc
    load_ref[...] += load_acc


@functools.partial(jax.jit, static_argnames=("interpret",))
def kernel(tokens, spatial_shape, gate_W, gate_b, W_down, b_down, W_up, b_up,
           interpret=False):
    del spatial_shape
    gb2 = gate_b.reshape(1, E)
    bd3 = b_down.reshape(E, 1, R)
    bu3 = b_up.reshape(E, 1, D)
    wd_bf = W_down.astype(jnp.bfloat16)
    wu_bf = W_up.astype(jnp.bfloat16)

    out, logits3, sel3, ew3, imp, load = pl.pallas_call(
        _fused_kernel,
        grid=(B // SPP,),
        in_specs=[
            pl.BlockSpec((SPP, T, D), lambda b: (b, 0, 0)),
            pl.BlockSpec((D, E), lambda b: (0, 0)),
            pl.BlockSpec((1, E), lambda b: (0, 0)),
            pl.BlockSpec((E, D, R), lambda b: (0, 0, 0)),
            pl.BlockSpec((E, 1, R), lambda b: (0, 0, 0)),
            pl.BlockSpec((E, R, D), lambda b: (0, 0, 0)),
            pl.BlockSpec((E, 1, D), lambda b: (0, 0, 0)),
        ],
        out_specs=[
            pl.BlockSpec((SPP, T, D), lambda b: (b, 0, 0)),
            pl.BlockSpec((SPP, 1, E), lambda b: (b, 0, 0)),
            pl.BlockSpec((SPP, 1, E), lambda b: (b, 0, 0)),
            pl.BlockSpec((SPP, 1, E), lambda b: (b, 0, 0)),
            pl.BlockSpec((1, E), lambda b: (0, 0)),
            pl.BlockSpec((1, E), lambda b: (0, 0)),
        ],
        out_shape=[
            jax.ShapeDtypeStruct((B, T, D), jnp.float32),
            jax.ShapeDtypeStruct((B, 1, E), jnp.float32),
            jax.ShapeDtypeStruct((B, 1, E), jnp.int32),
            jax.ShapeDtypeStruct((B, 1, E), jnp.float32),
            jax.ShapeDtypeStruct((1, E), jnp.float32),
            jax.ShapeDtypeStruct((1, E), jnp.float32),
        ],
        interpret=interpret,
    )(tokens, gate_W, gb2, wd_bf, bd3, wu_bf, bu3)

    expert_weights = ew3.reshape(B, E)

    router_logits = logits3.reshape(B, E)
    selected_experts = sel3[:, 0, :1]
    return (out, router_logits, selected_experts, expert_weights,
            imp.reshape(E), load.reshape(E))


# speedup vs baseline: 1.4272x; 1.4272x over previous
import functools

import jax
import jax.numpy as jnp
from jax.experimental import pallas as pl
from jax.experimental.pallas import tpu as pltpu

B, T, D = 64, 576, 768
E, R = 8, 192
SPP = 4


def _fused_kernel(tok_ref, gw_ref, gb_ref, wd_ref, bd_ref, wu_ref, bu_ref,
                  out_ref, logits_ref, sel_ref, ew_ref, imp_ref, load_ref):
    g = pl.program_id(0)
    lane = jax.lax.broadcasted_iota(jnp.int32, (1, E), 1)
    imp_acc = jnp.zeros((1, E), jnp.float32)
    load_acc = jnp.zeros((1, E), jnp.float32)

    for i in range(SPP):
        x = tok_ref[i]

        pooled = jnp.mean(x, axis=0, keepdims=True)
        logits = jnp.dot(pooled, gw_ref[...],
                         preferred_element_type=jnp.float32) + gb_ref[...]
        m = jnp.max(logits, axis=-1, keepdims=True)
        ex = jnp.exp(logits - m)
        probs = ex / jnp.sum(ex, axis=-1, keepdims=True)
        top1 = jnp.max(probs)
        sel = jnp.argmax(probs, axis=-1)[0].astype(jnp.int32)

        wd = wd_ref[sel]
        wu = wu_ref[sel]
        bd = bd_ref[sel]
        bu = bu_ref[sel]
        xb = x.astype(jnp.bfloat16)
        h = jnp.dot(xb, wd, preferred_element_type=jnp.float32) + bd
        h = jax.nn.gelu(h.astype(jnp.bfloat16))
        y = jnp.dot(h, wu, preferred_element_type=jnp.float32) + bu
        out_ref[i] = top1 * (x + y)

        logits_ref[i] = logits
        sel_ref[i] = jnp.zeros((1, E), jnp.int32) + sel
        onehot = (lane == sel).astype(jnp.float32)
        ew_ref[i] = onehot * top1
        imp_acc += onehot * top1
        load_acc += onehot * (1.0 / B)

    @pl.when(g == 0)
    def _init():
        imp_ref[...] = jnp.zeros_like(imp_ref)
        load_ref[...] = jnp.zeros_like(load_ref)

    imp_ref[...] += imp_acc
    load_ref[...] += load_acc


@functools.partial(jax.jit, static_argnames=("interpret",))
def kernel(tokens, spatial_shape, gate_W, gate_b, W_down, b_down, W_up, b_up,
           interpret=False):
    del spatial_shape
    gb2 = gate_b.reshape(1, E)
    bd3 = b_down.reshape(E, 1, R)
    bu3 = b_up.reshape(E, 1, D)
    wd_bf = W_down.astype(jnp.bfloat16)
    wu_bf = W_up.astype(jnp.bfloat16)

    out, logits3, sel3, ew3, imp, load = pl.pallas_call(
        _fused_kernel,
        grid=(B // SPP,),
        in_specs=[
            pl.BlockSpec((SPP, T, D), lambda b: (b, 0, 0)),
            pl.BlockSpec((D, E), lambda b: (0, 0)),
            pl.BlockSpec((1, E), lambda b: (0, 0)),
            pl.BlockSpec((E, D, R), lambda b: (0, 0, 0)),
            pl.BlockSpec((E, 1, R), lambda b: (0, 0, 0)),
            pl.BlockSpec((E, R, D), lambda b: (0, 0, 0)),
            pl.BlockSpec((E, 1, D), lambda b: (0, 0, 0)),
        ],
        out_specs=[
            pl.BlockSpec((SPP, T, D), lambda b: (b, 0, 0)),
            pl.BlockSpec((SPP, 1, E), lambda b: (b, 0, 0)),
            pl.BlockSpec((SPP, 1, E), lambda b: (b, 0, 0)),
            pl.BlockSpec((SPP, 1, E), lambda b: (b, 0, 0)),
            pl.BlockSpec((1, E), lambda b: (0, 0)),
            pl.BlockSpec((1, E), lambda b: (0, 0)),
        ],
        out_shape=[
            jax.ShapeDtypeStruct((B, T, D), jnp.float32),
            jax.ShapeDtypeStruct((B, 1, E), jnp.float32),
            jax.ShapeDtypeStruct((B, 1, E), jnp.int32),
            jax.ShapeDtypeStruct((B, 1, E), jnp.float32),
            jax.ShapeDtypeStruct((1, E), jnp.float32),
            jax.ShapeDtypeStruct((1, E), jnp.float32),
        ],
        interpret=interpret,
    )(tokens, gate_W, gb2, wd_bf, bd3, wu_bf, bu3)

    expert_weights = ew3.reshape(B, E)

    router_logits = logits3.reshape(B, E)
    selected_experts = sel3[:, 0, :1]
    return (out, router_logits, selected_experts, expert_weights,
            imp.reshape(E), load.reshape(E))
